# baseline (speedup 1.0000x reference)
import functools

import jax
import jax.numpy as jnp
from jax import lax
from jax.experimental import pallas as pl
from jax.experimental.pallas import tpu as pltpu
from jax.experimental.pallas import tpu_sc as plsc

_ETA = 10
_K = 64
_N_ENTS = 1000000
_N_RELS = 1000
_B = 16384
_M = _B * (1 + _ETA)
_NC, _NS = 2, 16
_NW = _NC * _NS
_RPW = _M // _NW
_CH = 128
_NCHUNK = _RPW // _CH
_GRP = _CH // 16


def _sc_scores(s_ids, r_ids, o_ids, ent_emb, rel_emb):
    mesh = plsc.VectorSubcoreMesh(
        core_axis_name="c", subcore_axis_name="s",
        num_cores=_NC, num_subcores=_NS)

    @functools.partial(
        pl.kernel,
        mesh=mesh,
        compiler_params=pltpu.CompilerParams(
            needs_layout_passes=False, use_tc_tiling_on_sc=False),
        out_type=jax.ShapeDtypeStruct((_M,), jnp.float32),
        scratch_types=[
            pltpu.VMEM((_N_RELS, _K), jnp.float32),
            pltpu.VMEM((_RPW,), jnp.int32),
            pltpu.VMEM((_RPW,), jnp.int32),
            pltpu.VMEM((_RPW,), jnp.int32),
            pltpu.VMEM((_CH, _K), jnp.float32),
            pltpu.VMEM((_CH, _K), jnp.float32),
            pltpu.VMEM((_RPW,), jnp.float32),
            pltpu.SemaphoreType.DMA,
            pltpu.SemaphoreType.DMA,
        ],
    )
    def k(s_hbm, r_hbm, o_hbm, ent_hbm, rel_hbm, out_hbm,
          rel_v, sid_v, rid_v, oid_v, srow_v, orow_v, out_v, sem_s, sem_o):
        wid = lax.axis_index("s") * _NC + lax.axis_index("c")
        base = wid * _RPW
        pltpu.sync_copy(rel_hbm, rel_v)
        pltpu.sync_copy(s_hbm.at[pl.ds(base, _RPW)], sid_v)
        pltpu.sync_copy(r_hbm.at[pl.ds(base, _RPW)], rid_v)
        pltpu.sync_copy(o_hbm.at[pl.ds(base, _RPW)], oid_v)

        lane = lax.iota(jnp.int32, 16)

        def chunk_body(c):
            off = c * _CH
            cp_s = pltpu.async_copy(
                ent_hbm.at[sid_v.at[pl.ds(off, _CH)]], srow_v, sem_s)
            cp_o = pltpu.async_copy(
                ent_hbm.at[oid_v.at[pl.ds(off, _CH)]], orow_v, sem_o)
            cp_s.wait()
            cp_o.wait()

            def group_body(g):
                lrow = g * 16 + lane
                rid = rid_v[pl.ds(off + g * 16, 16)]
                acc = jnp.zeros((16,), jnp.float32)
                for d in range(_K):
                    dsplat = jnp.full((16,), d, jnp.int32)
                    sv = plsc.load_gather(srow_v, [lrow, dsplat])
                    ov = plsc.load_gather(orow_v, [lrow, dsplat])
                    pv = plsc.load_gather(rel_v, [rid, dsplat])
                    acc = acc + sv * pv * ov
                out_v[pl.ds(off + g * 16, 16)] = acc

            pl.loop(0, _GRP)(group_body)

        pl.loop(0, _NCHUNK)(chunk_body)
        pltpu.sync_copy(out_v, out_hbm.at[pl.ds(base, _RPW)])

    return k(s_ids, r_ids, o_ids, ent_emb, rel_emb)


def kernel(triples, ent_emb, rel_emb):
    t = triples.astype(jnp.int32)
    key = jax.random.key(42)
    dataset = jnp.tile(t, (_ETA, 1))
    m = dataset.shape[0]
    k1, k2 = jax.random.split(key)
    keep_subj = jax.random.randint(k1, (m,), 0, 2)
    keep_obj = 1 - keep_subj
    repl = jax.random.randint(k2, (m,), 0, _N_ENTS)
    csubj = keep_subj * dataset[:, 0] + keep_obj * repl
    cobj = keep_obj * dataset[:, 2] + keep_subj * repl

    s_ids = jnp.concatenate([t[:, 0], csubj]).astype(jnp.int32)
    r_ids = jnp.concatenate([t[:, 1], dataset[:, 1]]).astype(jnp.int32)
    o_ids = jnp.concatenate([t[:, 2], cobj]).astype(jnp.int32)

    scores = _sc_scores(s_ids, r_ids, o_ids, ent_emb, rel_emb)
    return scores[:_B], scores[_B:]


# speedup vs baseline: 1.5831x; 1.5831x over previous
import functools

import jax
import jax.numpy as jnp
from jax import lax
from jax.experimental import pallas as pl
from jax.experimental.pallas import tpu as pltpu
from jax.experimental.pallas import tpu_sc as plsc

_ETA = 10
_K = 64
_N_ENTS = 1000000
_N_RELS = 1000
_B = 16384
_M = _B * (1 + _ETA)
_NC, _NS = 2, 16
_NW = _NC * _NS
_RPW = _M // _NW
_CH = 128
_NCHUNK = _RPW // _CH
_GRP = _CH // 16


def _sc_scores(s_ids, r_ids, o_ids, ent_emb, rel_emb):
    mesh = plsc.VectorSubcoreMesh(
        core_axis_name="c", subcore_axis_name="s",
        num_cores=_NC, num_subcores=_NS)

    @functools.partial(
        pl.kernel,
        mesh=mesh,
        compiler_params=pltpu.CompilerParams(
            needs_layout_passes=False, use_tc_tiling_on_sc=False),
        out_type=jax.ShapeDtypeStruct((_M,), jnp.float32),
        scratch_types=[
            pltpu.VMEM((_N_RELS, _K), jnp.float32),
            pltpu.VMEM((_RPW,), jnp.int32),
            pltpu.VMEM((_RPW,), jnp.int32),
            pltpu.VMEM((_RPW,), jnp.int32),
            pltpu.VMEM((_CH, _K), jnp.float32),
            pltpu.VMEM((_CH, _K), jnp.float32),
            pltpu.VMEM((16, 17), jnp.float32),
            pltpu.VMEM((_RPW,), jnp.float32),
            pltpu.SemaphoreType.DMA,
            pltpu.SemaphoreType.DMA,
        ],
    )
    def k(s_hbm, r_hbm, o_hbm, ent_hbm, rel_hbm, out_hbm,
          rel_v, sid_v, rid_v, oid_v, srow_v, orow_v, pad_v, out_v,
          sem_s, sem_o):
        wid = lax.axis_index("s") * _NC + lax.axis_index("c")
        base = wid * _RPW
        pltpu.sync_copy(rel_hbm, rel_v)
        pltpu.sync_copy(s_hbm.at[pl.ds(base, _RPW)], sid_v)
        pltpu.sync_copy(r_hbm.at[pl.ds(base, _RPW)], rid_v)
        pltpu.sync_copy(o_hbm.at[pl.ds(base, _RPW)], oid_v)

        lane = lax.iota(jnp.int32, 16)

        def chunk_body(c):
            off = c * _CH
            cp_s = pltpu.async_copy(
                ent_hbm.at[sid_v.at[pl.ds(off, _CH)]], srow_v, sem_s)
            cp_o = pltpu.async_copy(
                ent_hbm.at[oid_v.at[pl.ds(off, _CH)]], orow_v, sem_o)
            cp_s.wait()
            cp_o.wait()

            def group_body(g):
                grow = g * 16
                ridvec = rid_v[pl.ds(off + grow, 16)]
                for r in range(16):
                    rid = ridvec[r]
                    acc = None
                    for cc in range(_K // 16):
                        sv = srow_v[grow + r, pl.ds(cc * 16, 16)]
                        pv = rel_v[rid, pl.ds(cc * 16, 16)]
                        ov = orow_v[grow + r, pl.ds(cc * 16, 16)]
                        t = sv * pv * ov
                        acc = t if acc is None else acc + t
                    pad_v[r, pl.ds(0, 16)] = acc
                tot = None
                for d in range(16):
                    dsplat = jnp.full((16,), d, jnp.int32)
                    col = plsc.load_gather(pad_v, [lane, dsplat])
                    tot = col if tot is None else tot + col
                out_v[pl.ds(off + grow, 16)] = tot

            pl.loop(0, _GRP)(group_body)

        pl.loop(0, _NCHUNK)(chunk_body)
        pltpu.sync_copy(out_v, out_hbm.at[pl.ds(base, _RPW)])

    return k(s_ids, r_ids, o_ids, ent_emb, rel_emb)


def kernel(triples, ent_emb, rel_emb):
    t = triples.astype(jnp.int32)
    key = jax.random.key(42)
    dataset = jnp.tile(t, (_ETA, 1))
    m = dataset.shape[0]
    k1, k2 = jax.random.split(key)
    keep_subj = jax.random.randint(k1, (m,), 0, 2)
    keep_obj = 1 - keep_subj
    repl = jax.random.randint(k2, (m,), 0, _N_ENTS)
    csubj = keep_subj * dataset[:, 0] + keep_obj * repl
    cobj = keep_obj * dataset[:, 2] + keep_subj * repl

    s_ids = jnp.concatenate([t[:, 0], csubj]).astype(jnp.int32)
    r_ids = jnp.concatenate([t[:, 1], dataset[:, 1]]).astype(jnp.int32)
    o_ids = jnp.concatenate([t[:, 2], cobj]).astype(jnp.int32)

    scores = _sc_scores(s_ids, r_ids, o_ids, ent_emb, rel_emb)
    return scores[:_B], scores[_B:]


# speedup vs baseline: 1.6970x; 1.0720x over previous
import functools

import jax
import jax.numpy as jnp
from jax import lax
from jax.experimental import pallas as pl
from jax.experimental.pallas import tpu as pltpu
from jax.experimental.pallas import tpu_sc as plsc

_ETA = 10
_K = 64
_N_ENTS = 1000000
_N_RELS = 1000
_B = 16384
_M = _B * (1 + _ETA)
_NC, _NS = 2, 16
_NW = _NC * _NS
_RPW = _M // _NW
_CH = 128
_NCHUNK = _RPW // _CH
_GRP = _CH // 16


def _tree_sum(vals):
    vals = list(vals)
    while len(vals) > 1:
        nxt = [a + b for a, b in zip(vals[::2], vals[1::2])]
        if len(vals) % 2:
            nxt.append(vals[-1])
        vals = nxt
    return vals[0]


def _sc_scores(s_ids, r_ids, o_ids, ent_emb, rel_emb):
    mesh = plsc.VectorSubcoreMesh(
        core_axis_name="c", subcore_axis_name="s",
        num_cores=_NC, num_subcores=_NS)

    @functools.partial(
        pl.kernel,
        mesh=mesh,
        compiler_params=pltpu.CompilerParams(
            needs_layout_passes=False, use_tc_tiling_on_sc=False),
        out_type=jax.ShapeDtypeStruct((_M,), jnp.float32),
        scratch_types=[
            pltpu.VMEM((_N_RELS, _K), jnp.float32),
            pltpu.VMEM((_RPW,), jnp.int32),
            pltpu.VMEM((_RPW,), jnp.int32),
            pltpu.VMEM((_RPW,), jnp.int32),
            pltpu.VMEM((2, _CH, _K), jnp.float32),
            pltpu.VMEM((2, _CH, _K), jnp.float32),
            pltpu.VMEM((16, 17), jnp.float32),
            pltpu.VMEM((_RPW,), jnp.float32),
            pltpu.SemaphoreType.DMA,
            pltpu.SemaphoreType.DMA,
        ],
    )
    def k(s_hbm, r_hbm, o_hbm, ent_hbm, rel_hbm, out_hbm,
          rel_v, sid_v, rid_v, oid_v, srow_v, orow_v, pad_v, out_v,
          sem0, sem1):
        wid = lax.axis_index("s") * _NC + lax.axis_index("c")
        base = wid * _RPW
        pltpu.sync_copy(rel_hbm, rel_v)
        pltpu.sync_copy(s_hbm.at[pl.ds(base, _RPW)], sid_v)
        pltpu.sync_copy(r_hbm.at[pl.ds(base, _RPW)], rid_v)
        pltpu.sync_copy(o_hbm.at[pl.ds(base, _RPW)], oid_v)

        lane = lax.iota(jnp.int32, 16)
        sems = (sem0, sem1)

        def start(c, b):
            off = c * _CH
            pltpu.async_copy(
                ent_hbm.at[sid_v.at[pl.ds(off, _CH)]], srow_v.at[b], sems[b])
            pltpu.async_copy(
                ent_hbm.at[oid_v.at[pl.ds(off, _CH)]], orow_v.at[b], sems[b])

        def wait(b):
            pltpu.make_async_copy(
                ent_hbm.at[sid_v.at[pl.ds(0, _CH)]], srow_v.at[b],
                sems[b]).wait()
            pltpu.make_async_copy(
                ent_hbm.at[oid_v.at[pl.ds(0, _CH)]], orow_v.at[b],
                sems[b]).wait()

        def compute(c, b):
            off = c * _CH
            sb = srow_v.at[b]
            ob = orow_v.at[b]

            def group_body(g):
                grow = g * 16
                ridvec = rid_v[pl.ds(off + grow, 16)]
                for r in range(16):
                    rid = ridvec[r]
                    terms = []
                    for cc in range(_K // 16):
                        sv = sb[grow + r, pl.ds(cc * 16, 16)]
                        pv = rel_v[rid, pl.ds(cc * 16, 16)]
                        ov = ob[grow + r, pl.ds(cc * 16, 16)]
                        terms.append(sv * pv * ov)
                    pad_v[r, pl.ds(0, 16)] = _tree_sum(terms)
                cols = []
                for d in range(16):
                    dsplat = jnp.full((16,), d, jnp.int32)
                    cols.append(plsc.load_gather(pad_v, [lane, dsplat]))
                out_v[pl.ds(off + grow, 16)] = _tree_sum(cols)

            pl.loop(0, _GRP)(group_body)

        start(0, 0)
        start(1, 1)

        def chunk_pair(c):
            for b in range(2):
                cur = c + b
                wait(b)
                compute(cur, b)

                @pl.when(cur + 2 < _NCHUNK)
                def _():
                    start(cur + 2, b)

        pl.loop(0, _NCHUNK, step=2)(chunk_pair)
        pltpu.sync_copy(out_v, out_hbm.at[pl.ds(base, _RPW)])

    return k(s_ids, r_ids, o_ids, ent_emb, rel_emb)


def kernel(triples, ent_emb, rel_emb):
    t = triples.astype(jnp.int32)
    key = jax.random.key(42)
    dataset = jnp.tile(t, (_ETA, 1))
    m = dataset.shape[0]
    k1, k2 = jax.random.split(key)
    keep_subj = jax.random.randint(k1, (m,), 0, 2)
    keep_obj = 1 - keep_subj
    repl = jax.random.randint(k2, (m,), 0, _N_ENTS)
    csubj = keep_subj * dataset[:, 0] + keep_obj * repl
    cobj = keep_obj * dataset[:, 2] + keep_subj * repl

    s_ids = jnp.concatenate([t[:, 0], csubj]).astype(jnp.int32)
    r_ids = jnp.concatenate([t[:, 1], dataset[:, 1]]).astype(jnp.int32)
    o_ids = jnp.concatenate([t[:, 2], cobj]).astype(jnp.int32)

    scores = _sc_scores(s_ids, r_ids, o_ids, ent_emb, rel_emb)
    return scores[:_B], scores[_B:]


# speedup vs baseline: 1.7200x; 1.0135x over previous
import functools

import jax
import jax.numpy as jnp
from jax import lax
from jax.experimental import pallas as pl
from jax.experimental.pallas import tpu as pltpu
from jax.experimental.pallas import tpu_sc as plsc

_ETA = 10
_K = 64
_N_ENTS = 1000000
_N_RELS = 1000
_B = 16384
_M = _B * (1 + _ETA)
_NC, _NS = 2, 16
_NW = _NC * _NS
_OPW = _B // _NW
_CH = 128
_NBLK = _OPW // _CH


def _tree_sum(vals):
    vals = list(vals)
    while len(vals) > 1:
        nxt = [a + b for a, b in zip(vals[::2], vals[1::2])]
        if len(vals) % 2:
            nxt.append(vals[-1])
        vals = nxt
    return vals[0]


def _sc_scores(subj2, rel2, obj2, repl3, keep3, ent_emb, rel_emb):
    mesh = plsc.VectorSubcoreMesh(
        core_axis_name="c", subcore_axis_name="s",
        num_cores=_NC, num_subcores=_NS)

    @functools.partial(
        pl.kernel,
        mesh=mesh,
        compiler_params=pltpu.CompilerParams(
            needs_layout_passes=False, use_tc_tiling_on_sc=False),
        out_type=jax.ShapeDtypeStruct((_M,), jnp.float32),
        scratch_types=[
            pltpu.VMEM((_OPW,), jnp.int32),
            pltpu.VMEM((_OPW,), jnp.int32),
            pltpu.VMEM((_OPW,), jnp.int32),
            pltpu.VMEM((_ETA, _OPW), jnp.int32),
            pltpu.VMEM((_ETA, _OPW), jnp.int32),
            pltpu.VMEM((_CH, _K), jnp.float32),
            pltpu.VMEM((_CH, _K), jnp.float32),
            pltpu.VMEM((_CH, _K), jnp.float32),
            pltpu.VMEM((2, _CH, _K), jnp.float32),
            pltpu.VMEM((_CH, 2 * _K), jnp.float32),
            pltpu.VMEM((16, 17), jnp.float32),
            pltpu.VMEM((_OPW,), jnp.float32),
            pltpu.VMEM((_ETA, _OPW), jnp.float32),
            pltpu.SemaphoreType.DMA,
            pltpu.SemaphoreType.DMA,
            pltpu.SemaphoreType.DMA,
        ],
    )
    def k(s_hbm, r_hbm, o_hbm, repl_hbm, keep_hbm, ent_hbm, relt_hbm, out_hbm,
          sid_v, rid_v, oid_v, repl_v, keep_v,
          sbuf, obuf, pbuf, rbuf, uv_v, pad_v, inp_v, cor_v,
          sem_g, sem_r0, sem_r1):
        wid = lax.axis_index("s") * _NC + lax.axis_index("c")
        pltpu.sync_copy(s_hbm.at[wid], sid_v)
        pltpu.sync_copy(r_hbm.at[wid], rid_v)
        pltpu.sync_copy(o_hbm.at[wid], oid_v)
        pltpu.sync_copy(repl_hbm.at[wid], repl_v)
        pltpu.sync_copy(keep_hbm.at[wid], keep_v)

        lane = lax.iota(jnp.int32, 16)
        rsems = (sem_r0, sem_r1)

        def start_orig(blk):
            off = blk * _CH
            pltpu.async_copy(
                ent_hbm.at[sid_v.at[pl.ds(off, _CH)]], sbuf, sem_g)
            pltpu.async_copy(
                ent_hbm.at[oid_v.at[pl.ds(off, _CH)]], obuf, sem_g)
            pltpu.async_copy(
                relt_hbm.at[rid_v.at[pl.ds(off, _CH)]], pbuf, sem_g)

        def wait_orig():
            pltpu.make_async_copy(
                ent_hbm.at[sid_v.at[pl.ds(0, _CH)]], sbuf, sem_g).wait()
            pltpu.make_async_copy(
                ent_hbm.at[oid_v.at[pl.ds(0, _CH)]], obuf, sem_g).wait()
            pltpu.make_async_copy(
                relt_hbm.at[rid_v.at[pl.ds(0, _CH)]], pbuf, sem_g).wait()

        def start_repl(blk, kk, b):
            pltpu.async_copy(
                ent_hbm.at[repl_v.at[kk, pl.ds(blk * _CH, _CH)]],
                rbuf.at[b], rsems[b])

        def wait_repl(b):
            pltpu.make_async_copy(
                ent_hbm.at[repl_v.at[0, pl.ds(0, _CH)]],
                rbuf.at[b], rsems[b]).wait()

        def transpose_sum(cols_src):
            cols = []
            for d in range(16):
                dsplat = jnp.full((16,), d, jnp.int32)
                cols.append(plsc.load_gather(cols_src, [lane, dsplat]))
            return _tree_sum(cols)

        def block_body(blk):
            boff = blk * _CH
            wait_orig()
            start_repl(blk, 0, 0)
            start_repl(blk, 1, 1)

            def uv_group(g):
                grow = g * 16
                for r in range(16):
                    terms = []
                    for cc in range(_K // 16):
                        sv = sbuf[grow + r, pl.ds(cc * 16, 16)]
                        pv = pbuf[grow + r, pl.ds(cc * 16, 16)]
                        ov = obuf[grow + r, pl.ds(cc * 16, 16)]
                        u = sv * pv
                        v = pv * ov
                        uv_v[grow + r, pl.ds(cc * 16, 16)] = u
                        uv_v[grow + r, pl.ds(_K + cc * 16, 16)] = v
                        terms.append(u * ov)
                    pad_v[r, pl.ds(0, 16)] = _tree_sum(terms)
                inp_v[pl.ds(boff + grow, 16)] = transpose_sum(pad_v)

            pl.loop(0, _CH // 16)(uv_group)

            @pl.when(blk + 1 < _NBLK)
            def _():
                start_orig(blk + 1)

            def corr_pair(kk0):
                for b in range(2):
                    kk = kk0 + b
                    wait_repl(b)
                    rb = rbuf.at[b]

                    def corr_group(g):
                        grow = g * 16
                        kvec = keep_v[kk, pl.ds(boff + grow, 16)]
                        offv = (1 - kvec) * _K
                        for r in range(16):
                            off_r = offv[r]
                            terms = []
                            for cc in range(_K // 16):
                                w = uv_v[grow + r, pl.ds(off_r + cc * 16, 16)]
                                rv = rb[grow + r, pl.ds(cc * 16, 16)]
                                terms.append(w * rv)
                            pad_v[r, pl.ds(0, 16)] = _tree_sum(terms)
                        cor_v[kk, pl.ds(boff + grow, 16)] = transpose_sum(pad_v)

                    pl.loop(0, _CH // 16)(corr_group)

                    @pl.when(kk + 2 < _ETA)
                    def _():
                        start_repl(blk, kk + 2, b)

            pl.loop(0, _ETA, step=2)(corr_pair)

        start_orig(0)
        pl.loop(0, _NBLK)(block_body)

        pltpu.sync_copy(inp_v, out_hbm.at[pl.ds(wid * _OPW, _OPW)])
        for kk in range(_ETA):
            pltpu.sync_copy(
                cor_v.at[kk],
                out_hbm.at[pl.ds((kk + 1) * _B + wid * _OPW, _OPW)])

    return k(subj2, rel2, obj2, repl3, keep3, ent_emb, rel_emb)


def kernel(triples, ent_emb, rel_emb):
    t = triples.astype(jnp.int32)
    key = jax.random.key(42)
    m = _B * _ETA
    k1, k2 = jax.random.split(key)
    keep_subj = jax.random.randint(k1, (m,), 0, 2)
    repl = jax.random.randint(k2, (m,), 0, _N_ENTS)

    subj2 = t[:, 0].reshape(_NW, _OPW)
    rel2 = t[:, 1].reshape(_NW, _OPW)
    obj2 = t[:, 2].reshape(_NW, _OPW)
    repl3 = repl.reshape(_ETA, _NW, _OPW).transpose(1, 0, 2)
    keep3 = keep_subj.reshape(_ETA, _NW, _OPW).transpose(1, 0, 2)

    scores = _sc_scores(subj2, rel2, obj2, repl3, keep3, ent_emb, rel_emb)
    return scores[:_B], scores[_B:]
